# baseline (speedup 1.0000x reference)
import numpy as np
import jax
import jax.numpy as jnp
from jax.experimental import pallas as pl
from jax.experimental.pallas import tpu as pltpu

D_MODEL = 1024
D_HALF = D_MODEL // 2
_SPINE = (0.0, 2.0, 4.0, 12.0, 36.0, 104.0, 304.0, 888.0, 2592.0, 7568.0)
_TOK_BLOCK = 2048

_S1 = np.float32(3.1392769820075657)
_S3 = np.float32(-5.136389542380186)
_S5 = np.float32(2.4346688558908527)
_S7 = np.float32(-0.43780054636746224)

_DIMS = (((0,), (0,)), ((), ()))


def _pe_kernel(pos_ref, posc_ref, df_ref, ph_ref, wt_ref, b_ref, g_ref,
               be_ref, out_ref):
    pos = pos_ref[...]
    df = df_ref[...]
    ph = ph_ref[...]
    u = pos * df + ph
    fr = u - jnp.floor(u)
    t = 2.0 * fr - 1.0
    t2 = t * t
    pe = t * (_S1 + t2 * (_S3 + t2 * (_S5 + t2 * _S7)))

    inv_d = np.float32(1.0 / D_HALF)
    wt = wt_ref[...]
    wc = wt - jnp.sum(wt, axis=1, keepdims=True) * inv_d
    bc = b_ref[...] - jnp.sum(b_ref[...], axis=1, keepdims=True) * inv_d
    rows = jnp.concatenate([wc, bc], axis=0)

    def gdot(i, j):
        return jnp.sum(rows[i:i + 1, :] * rows[j:j + 1, :],
                       axis=1, keepdims=True) * inv_d

    g00, g11, g22, g33 = gdot(0, 0), gdot(1, 1), gdot(2, 2), gdot(3, 3)
    g01, g02, g12 = 2.0 * gdot(0, 1), 2.0 * gdot(0, 2), 2.0 * gdot(1, 2)
    g03, g13, g23 = 2.0 * gdot(0, 3), 2.0 * gdot(1, 3), 2.0 * gdot(2, 3)

    wrows = jnp.concatenate([rows * g_ref[...], be_ref[...]], axis=0)
    wh = wrows.astype(jnp.bfloat16)
    wl = (wrows - wh.astype(jnp.float32)).astype(jnp.bfloat16)

    posc = posc_ref[0]
    lvl = jnp.zeros_like(posc)
    floor = jnp.zeros_like(posc)
    ceil = jnp.zeros_like(posc)
    prev = 0.0
    prev_ge = None
    for s in _SPINE:
        ge = (posc >= s).astype(jnp.float32)
        lvl = lvl + ge
        floor = floor + (s - prev) * ge
        if prev_ge is not None:
            ceil = ceil + (s - prev) * prev_ge
        prev, prev_ge = s, ge
    left = posc - floor
    right = jnp.maximum(ceil - posc, 0.0)

    var = (left * (g00 * left + g01 * right + g02 * lvl + g03)
           + right * (g11 * right + g12 * lvl + g13)
           + lvl * (g22 * lvl + g23)
           + g33)
    s_ = jax.lax.rsqrt(var + 1e-5)

    ones = jnp.ones_like(posc)
    a = jnp.concatenate(
        [left * s_, right * s_, lvl * s_, s_, ones], axis=0)
    ah = a.astype(jnp.bfloat16)
    al = (a - ah.astype(jnp.float32)).astype(jnp.bfloat16)
    hn = (jax.lax.dot_general(ah, wh, _DIMS, preferred_element_type=jnp.float32)
          + jax.lax.dot_general(ah, wl, _DIMS,
                                preferred_element_type=jnp.float32)
          + jax.lax.dot_general(al, wh, _DIMS,
                                preferred_element_type=jnp.float32))
    lat = 0.5 * hn * (1.0 + jax.lax.erf(hn * np.float32(1.0 / np.sqrt(2.0))))

    out_ref[:, :D_HALF] = pe
    out_ref[:, D_HALF:] = lat


def kernel(positions, W, b, gamma, beta):
    B, S = positions.shape
    N = B * S
    d_half = W.shape[0]
    T = _TOK_BLOCK
    pos_f = positions.reshape(N, 1).astype(jnp.float32)
    pos_c = positions.reshape(N // T, 1, T).astype(jnp.float32)

    div = np.exp(np.arange(0, d_half, 2, dtype=np.float64)
                 * -(np.log(10000.0) / d_half))
    df = jnp.asarray(np.repeat(div / (2.0 * np.pi), 2).reshape(1, d_half),
                     dtype=jnp.float32)
    ph = jnp.asarray(np.tile(np.array([0.5, 0.75]), d_half // 2)
                     .reshape(1, d_half), dtype=jnp.float32)

    grid = (N // T,)
    out = pl.pallas_call(
        _pe_kernel,
        grid=grid,
        in_specs=[
            pl.BlockSpec((T, 1), lambda i: (i, 0)),
            pl.BlockSpec((1, 1, T), lambda i: (i, 0, 0)),
            pl.BlockSpec((1, d_half), lambda i: (0, 0)),
            pl.BlockSpec((1, d_half), lambda i: (0, 0)),
            pl.BlockSpec((3, d_half), lambda i: (0, 0)),
            pl.BlockSpec((1, d_half), lambda i: (0, 0)),
            pl.BlockSpec((1, d_half), lambda i: (0, 0)),
            pl.BlockSpec((1, d_half), lambda i: (0, 0)),
        ],
        out_specs=pl.BlockSpec((T, 2 * d_half), lambda i: (i, 0)),
        out_shape=jax.ShapeDtypeStruct((N, 2 * d_half), jnp.float32),
    )(pos_f, pos_c, df, ph, W.T, b.reshape(1, d_half),
      gamma.reshape(1, d_half), beta.reshape(1, d_half))
    return out.reshape(B, S, 2 * d_half)


# speedup vs baseline: 1.4241x; 1.4241x over previous
import numpy as np
import jax
import jax.numpy as jnp
from jax.experimental import pallas as pl
from jax.experimental.pallas import tpu as pltpu

D_MODEL = 1024
D_HALF = D_MODEL // 2
_SPINE = (0.0, 2.0, 4.0, 12.0, 36.0, 104.0, 304.0, 888.0, 2592.0, 7568.0)
_TOK_BLOCK = 1024

_S1 = np.float32(3.1392769820075657)
_S3 = np.float32(-5.136389542380186)
_S5 = np.float32(2.4346688558908527)
_S7 = np.float32(-0.43780054636746224)

_DIMS = (((0,), (0,)), ((), ()))


def _pe_kernel(posc_ref, df_ref, ph_ref, wt_ref, b_ref, g_ref,
               be_ref, out_ref):
    df = df_ref[...]
    ph = ph_ref[...]
    zero = jnp.zeros_like(df)
    d1 = df.astype(jnp.bfloat16)
    r1 = df - d1.astype(jnp.float32)
    d2 = r1.astype(jnp.bfloat16)
    d3 = (r1 - d2.astype(jnp.float32)).astype(jnp.bfloat16)
    phb = ph.astype(jnp.bfloat16)

    inv_d = np.float32(1.0 / D_HALF)
    wt = wt_ref[...]
    wc = wt - jnp.sum(wt, axis=1, keepdims=True) * inv_d
    bc = b_ref[...] - jnp.sum(b_ref[...], axis=1, keepdims=True) * inv_d
    rows = jnp.concatenate([wc, bc], axis=0)

    def gdot(i, j):
        return jnp.sum(rows[i:i + 1, :] * rows[j:j + 1, :],
                       axis=1, keepdims=True) * inv_d

    g00, g11, g22, g33 = gdot(0, 0), gdot(1, 1), gdot(2, 2), gdot(3, 3)
    g01, g02, g12 = 2.0 * gdot(0, 1), 2.0 * gdot(0, 2), 2.0 * gdot(1, 2)
    g03, g13, g23 = 2.0 * gdot(0, 3), 2.0 * gdot(1, 3), 2.0 * gdot(2, 3)

    wrows = jnp.concatenate([rows * g_ref[...], be_ref[...]], axis=0)
    wh = wrows.astype(jnp.bfloat16)
    wl = (wrows - wh.astype(jnp.float32)).astype(jnp.bfloat16)

    posc = posc_ref[0]
    lvl = jnp.zeros_like(posc)
    floor = jnp.zeros_like(posc)
    ceil = jnp.zeros_like(posc)
    prev = 0.0
    prev_ge = None
    for s in _SPINE:
        ge = (posc >= s).astype(jnp.float32)
        lvl = lvl + ge
        floor = floor + (s - prev) * ge
        if prev_ge is not None:
            ceil = ceil + (s - prev) * prev_ge
        prev, prev_ge = s, ge
    left = posc - floor
    right = jnp.maximum(ceil - posc, 0.0)

    var = (left * (g00 * left + g01 * right + g02 * lvl + g03)
           + right * (g11 * right + g12 * lvl + g13)
           + lvl * (g22 * lvl + g23)
           + g33)
    s_ = jax.lax.rsqrt(var + 1e-5)

    ones = jnp.ones_like(posc)
    a = jnp.concatenate(
        [left * s_, right * s_, lvl * s_, s_, ones], axis=0)
    ah = a.astype(jnp.bfloat16)
    al = (a - ah.astype(jnp.float32)).astype(jnp.bfloat16)

    p1 = posc.astype(jnp.bfloat16)
    p2 = (posc - p1.astype(jnp.float32)).astype(jnp.bfloat16)
    onesb = ones.astype(jnp.bfloat16)
    A = jnp.concatenate([p1, p1, p1, p2, p2, onesb, ah, ah, al],
                        axis=0)
    zb = zero.astype(jnp.bfloat16)
    wz = jnp.zeros_like(wh)

    def pad_l(x):
        return jnp.concatenate([x, zb], axis=1)

    B = jnp.concatenate(
        [pad_l(d1), pad_l(d2), pad_l(d3), pad_l(d1), pad_l(d2), pad_l(phb),
         jnp.concatenate([wz, wh], axis=1),
         jnp.concatenate([wz, wl], axis=1),
         jnp.concatenate([wz, wh], axis=1)], axis=0)
    R = jax.lax.dot_general(A, B, _DIMS,
                            preferred_element_type=jnp.float32)
    u = R[:, :D_HALF]
    hn = R[:, D_HALF:]

    fr = u - jnp.floor(u)
    t = 2.0 * fr - 1.0
    t2 = t * t
    pe = t * (_S1 + t2 * (_S3 + t2 * (_S5 + t2 * _S7)))
    lat = 0.5 * hn * (1.0 + jax.lax.erf(hn * np.float32(1.0 / np.sqrt(2.0))))

    out_ref[:, :D_HALF] = pe
    out_ref[:, D_HALF:] = lat


def kernel(positions, W, b, gamma, beta):
    B, S = positions.shape
    N = B * S
    d_half = W.shape[0]
    T = _TOK_BLOCK
    pos_c = positions.reshape(N // T, 1, T).astype(jnp.float32)

    div = np.exp(np.arange(0, d_half, 2, dtype=np.float64)
                 * -(np.log(10000.0) / d_half))
    df = jnp.asarray(np.repeat(div / (2.0 * np.pi), 2).reshape(1, d_half),
                     dtype=jnp.float32)
    ph = jnp.asarray(np.tile(np.array([0.5, 0.75]), d_half // 2)
                     .reshape(1, d_half), dtype=jnp.float32)

    grid = (N // T,)
    out = pl.pallas_call(
        _pe_kernel,
        grid=grid,
        in_specs=[
            pl.BlockSpec((1, 1, T), lambda i: (i, 0, 0)),
            pl.BlockSpec((1, d_half), lambda i: (0, 0)),
            pl.BlockSpec((1, d_half), lambda i: (0, 0)),
            pl.BlockSpec((3, d_half), lambda i: (0, 0)),
            pl.BlockSpec((1, d_half), lambda i: (0, 0)),
            pl.BlockSpec((1, d_half), lambda i: (0, 0)),
            pl.BlockSpec((1, d_half), lambda i: (0, 0)),
        ],
        out_specs=pl.BlockSpec((T, 2 * d_half), lambda i: (i, 0)),
        out_shape=jax.ShapeDtypeStruct((N, 2 * d_half), jnp.float32),
    )(pos_c, df, ph, W.T, b.reshape(1, d_half),
      gamma.reshape(1, d_half), beta.reshape(1, d_half))
    return out.reshape(B, S, 2 * d_half)


# speedup vs baseline: 1.4852x; 1.0429x over previous
import numpy as np
import jax
import jax.numpy as jnp
from jax.experimental import pallas as pl
from jax.experimental.pallas import tpu as pltpu

D_MODEL = 1024
D_HALF = D_MODEL // 2
_SPINE = (0.0, 2.0, 4.0, 12.0, 36.0, 104.0, 304.0, 888.0, 2592.0, 7568.0)
_TOK_BLOCK = 2048

_S1 = np.float32(3.1392769820075657)
_S3 = np.float32(-5.136389542380186)
_S5 = np.float32(2.4346688558908527)
_S7 = np.float32(-0.43780054636746224)

_DIMS = (((0,), (0,)), ((), ()))


def _pe_kernel(posc_ref, df_ref, ph_ref, wt_ref, b_ref, g_ref,
               be_ref, out_ref):
    df = df_ref[...]
    ph = ph_ref[...]
    zero = jnp.zeros_like(df)
    d1 = df.astype(jnp.bfloat16)
    r1 = df - d1.astype(jnp.float32)
    d2 = r1.astype(jnp.bfloat16)
    d3 = (r1 - d2.astype(jnp.float32)).astype(jnp.bfloat16)
    phb = ph.astype(jnp.bfloat16)

    inv_d = np.float32(1.0 / D_HALF)
    wt = wt_ref[...]
    wc = wt - jnp.sum(wt, axis=1, keepdims=True) * inv_d
    bc = b_ref[...] - jnp.sum(b_ref[...], axis=1, keepdims=True) * inv_d
    rows = jnp.concatenate([wc, bc], axis=0)

    def gdot(i, j):
        return jnp.sum(rows[i:i + 1, :] * rows[j:j + 1, :],
                       axis=1, keepdims=True) * inv_d

    g00, g11, g22, g33 = gdot(0, 0), gdot(1, 1), gdot(2, 2), gdot(3, 3)
    g01, g02, g12 = 2.0 * gdot(0, 1), 2.0 * gdot(0, 2), 2.0 * gdot(1, 2)
    g03, g13, g23 = 2.0 * gdot(0, 3), 2.0 * gdot(1, 3), 2.0 * gdot(2, 3)

    wrows = jnp.concatenate([rows * g_ref[...], be_ref[...]], axis=0)
    wh = wrows.astype(jnp.bfloat16)
    wl = (wrows - wh.astype(jnp.float32)).astype(jnp.bfloat16)

    posc = posc_ref[0]
    lvl = jnp.zeros_like(posc)
    floor = jnp.zeros_like(posc)
    ceil = jnp.zeros_like(posc)
    prev = 0.0
    prev_ge = None
    for s in _SPINE:
        ge = (posc >= s).astype(jnp.float32)
        lvl = lvl + ge
        floor = floor + (s - prev) * ge
        if prev_ge is not None:
            ceil = ceil + (s - prev) * prev_ge
        prev, prev_ge = s, ge
    left = posc - floor
    right = jnp.maximum(ceil - posc, 0.0)

    var = (left * (g00 * left + g01 * right + g02 * lvl + g03)
           + right * (g11 * right + g12 * lvl + g13)
           + lvl * (g22 * lvl + g23)
           + g33)
    s_ = jax.lax.rsqrt(var + 1e-5)

    ones = jnp.ones_like(posc)
    a = jnp.concatenate(
        [left * s_, right * s_, lvl * s_, s_, ones], axis=0)
    ah = a.astype(jnp.bfloat16)
    al = (a - ah.astype(jnp.float32)).astype(jnp.bfloat16)

    p1 = posc.astype(jnp.bfloat16)
    p2 = (posc - p1.astype(jnp.float32)).astype(jnp.bfloat16)
    onesb = ones.astype(jnp.bfloat16)
    A = jnp.concatenate([p1, p1, p1, p2, p2, onesb, ah, ah, al],
                        axis=0)
    zb = zero.astype(jnp.bfloat16)
    wz = jnp.zeros_like(wh)

    def pad_l(x):
        return jnp.concatenate([x, zb], axis=1)

    B = jnp.concatenate(
        [pad_l(d1), pad_l(d2), pad_l(d3), pad_l(d1), pad_l(d2), pad_l(phb),
         jnp.concatenate([wz, wh], axis=1),
         jnp.concatenate([wz, wl], axis=1),
         jnp.concatenate([wz, wh], axis=1)], axis=0)
    R = jax.lax.dot_general(A, B, _DIMS,
                            preferred_element_type=jnp.float32)
    u = R[:, :D_HALF]
    hn = R[:, D_HALF:]

    fr = u - jnp.floor(u)
    t = 2.0 * fr - 1.0
    t2 = t * t
    pe = t * (_S1 + t2 * (_S3 + t2 * (_S5 + t2 * _S7)))
    lat = 0.5 * hn * (1.0 + jax.lax.erf(hn * np.float32(1.0 / np.sqrt(2.0))))

    out_ref[:, :D_HALF] = pe
    out_ref[:, D_HALF:] = lat


def kernel(positions, W, b, gamma, beta):
    B, S = positions.shape
    N = B * S
    d_half = W.shape[0]
    T = _TOK_BLOCK
    pos_c = positions.reshape(N // T, 1, T).astype(jnp.float32)

    div = np.exp(np.arange(0, d_half, 2, dtype=np.float64)
                 * -(np.log(10000.0) / d_half))
    df = jnp.asarray(np.repeat(div / (2.0 * np.pi), 2).reshape(1, d_half),
                     dtype=jnp.float32)
    ph = jnp.asarray(np.tile(np.array([0.5, 0.75]), d_half // 2)
                     .reshape(1, d_half), dtype=jnp.float32)

    grid = (N // T,)
    out = pl.pallas_call(
        _pe_kernel,
        grid=grid,
        in_specs=[
            pl.BlockSpec((1, 1, T), lambda i: (i, 0, 0)),
            pl.BlockSpec((1, d_half), lambda i: (0, 0)),
            pl.BlockSpec((1, d_half), lambda i: (0, 0)),
            pl.BlockSpec((3, d_half), lambda i: (0, 0)),
            pl.BlockSpec((1, d_half), lambda i: (0, 0)),
            pl.BlockSpec((1, d_half), lambda i: (0, 0)),
            pl.BlockSpec((1, d_half), lambda i: (0, 0)),
        ],
        out_specs=pl.BlockSpec((T, 2 * d_half), lambda i: (i, 0)),
        out_shape=jax.ShapeDtypeStruct((N, 2 * d_half), jnp.float32),
    )(pos_c, df, ph, W.T, b.reshape(1, d_half),
      gamma.reshape(1, d_half), beta.reshape(1, d_half))
    return out.reshape(B, S, 2 * d_half)
